# baseline (speedup 1.0000x reference)
import functools

import jax
import jax.numpy as jnp
from jax import lax
from jax.experimental import pallas as pl
from jax.experimental.pallas import tpu as pltpu
from jax.experimental.pallas import tpu_sc as plsc

BATCH = 4096
HIST = 200
D_EMBED = 64
N_VOCAB = 1000000
NUM_CORES = 2
NUM_SUBCORES = 16
NW = NUM_CORES * NUM_SUBCORES
L = 16

TILE_COLS = N_VOCAB // 128
TAIL_R0 = TILE_COLS * 128
N_PAIR = N_VOCAB // 2

B_BLK = BATCH // NW


def _fmt_body(wt_hbm, tail_hbm, wpk_hbm, src_v, dst_v, tail_v, sem):
    wid = lax.axis_index("s") * NUM_CORES + lax.axis_index("c")

    lane = lax.iota(jnp.int32, L)
    lane_row = lane >> 1
    lane_col = (lane & 1) * 64

    def do_col(j):
        pltpu.sync_copy(wt_hbm.at[:, pl.ds(j * 128, 128)], src_v)

        def row_d(d, c):
            def seg_m(m, c2):
                v = src_v[d, pl.ds(m * L, L)]
                plsc.store_scatter(dst_v, [m * 8 + lane_row, lane_col + d], v)
                return c2

            return lax.fori_loop(0, 8, seg_m, c, unroll=True)

        lax.fori_loop(0, 64, row_d, 0)
        pltpu.sync_copy(dst_v, wpk_hbm.at[pl.ds(j * 64, 64)])

    def step(t, carry):
        j = wid + t * NW

        @pl.when(j < TILE_COLS)
        def _():
            do_col(j)

        return carry

    n_steps = (TILE_COLS + NW - 1) // NW
    lax.fori_loop(0, n_steps, step, 0)

    @pl.when(wid == 0)
    def _():
        pltpu.sync_copy(tail_hbm, tail_v)

        def row_q(q, c):
            rowv = lane_row * 0 + (q >> 1)
            colb = (q & 1) * 64

            def seg_m(m, c2):
                v = tail_v[q, pl.ds(m * L, L)]
                plsc.store_scatter(dst_v, [rowv, colb + m * L + lane], v)
                return c2

            return lax.fori_loop(0, 4, seg_m, c, unroll=True)

        lax.fori_loop(0, 64, row_q, 0)
        pltpu.sync_copy(dst_v.at[pl.ds(0, 32)], wpk_hbm.at[pl.ds(N_PAIR - 32, 32)])


def _gather_body(xt_hbm, wpk_hbm, out_hbm, idx_v, par_v, off_v, gbuf_v, obuf_v,
                 gsem, osem):
    wid = lax.axis_index("s") * NUM_CORES + lax.axis_index("c")
    b0 = wid * B_BLK

    pltpu.sync_copy(xt_hbm.at[:, pl.ds(b0, B_BLK)], idx_v)

    lane = lax.iota(jnp.int32, L)

    def conv_h(h, c):
        def seg_m(m, c2):
            r = idx_v[h, pl.ds(m * L, L)]
            par_v[h, pl.ds(m * L, L)] = (r & 1) * 64
            idx_v[h, pl.ds(m * L, L)] = r >> 1
            return c2

        return lax.fori_loop(0, 8, seg_m, c, unroll=True)

    lax.fori_loop(0, HIST, conv_h, 0)

    def wait_g(slot):
        pltpu.make_async_copy(
            wpk_hbm.at[idx_v.at[0]], gbuf_v.at[slot], gsem.at[slot]
        ).wait()

    def start_g(h, slot):
        pltpu.async_copy(
            wpk_hbm.at[idx_v.at[h]], gbuf_v.at[slot], gsem.at[slot]
        )

    def wait_o(slot):
        pltpu.make_async_copy(
            obuf_v.at[slot], out_hbm.at[0, :, pl.ds(b0, B_BLK)], osem.at[slot]
        ).wait()

    def start_o(h, slot):
        pltpu.async_copy(
            obuf_v.at[slot], out_hbm.at[h, :, pl.ds(b0, B_BLK)], osem.at[slot]
        )

    start_g(0, 0)
    start_g(1, 1)

    def step_hh(hh, carry):
        for slot in (0, 1):
            h = hh * 2 + slot
            wait_g(slot)

            @pl.when(h >= 2)
            def _():
                wait_o(slot)

            def copy_off(m, c):
                off_v[pl.ds(m * L, L)] = par_v[h, pl.ds(m * L, L)]
                return c

            lax.fori_loop(0, 8, copy_off, 0, unroll=True)

            def row_d(d, c):
                def seg_m(m, c2):
                    col = off_v[pl.ds(m * L, L)] + d
                    row = m * L + lane
                    obuf_v[slot, d, pl.ds(m * L, L)] = plsc.load_gather(
                        gbuf_v.at[slot], [row, col]
                    )
                    return c2

                return lax.fori_loop(0, 8, seg_m, c, unroll=True)

            lax.fori_loop(0, 64, row_d, 0)
            start_o(h, slot)

            @pl.when(h + 2 < HIST)
            def _():
                start_g(h + 2, slot)

        return carry

    lax.fori_loop(0, HIST // 2, step_hh, 0)
    wait_o(0)
    wait_o(1)


def kernel(x, W_E):
    mesh = plsc.VectorSubcoreMesh(core_axis_name="c", subcore_axis_name="s")
    wt = W_E.T
    tail = W_E[TAIL_R0:]

    wpk = pl.kernel(
        _fmt_body,
        mesh=mesh,
        out_type=jax.ShapeDtypeStruct((N_PAIR, 128), jnp.float32),
        scratch_types=[
            pltpu.VMEM((64, 128), jnp.float32),
            pltpu.VMEM((64, 128), jnp.float32),
            pltpu.VMEM((64, 64), jnp.float32),
            pltpu.SemaphoreType.DMA,
        ],
        compiler_params=pltpu.CompilerParams(use_tc_tiling_on_sc=True, needs_layout_passes=False),
    )(wt, tail)

    xt = x.T.astype(jnp.int32)
    out = pl.kernel(
        _gather_body,
        mesh=mesh,
        out_type=jax.ShapeDtypeStruct((HIST, D_EMBED, BATCH), jnp.float32),
        scratch_types=[
            pltpu.VMEM((HIST, B_BLK), jnp.int32),
            pltpu.VMEM((HIST, B_BLK), jnp.int32),
            pltpu.VMEM((B_BLK,), jnp.int32),
            pltpu.VMEM((2, B_BLK, 128), jnp.float32),
            pltpu.VMEM((2, D_EMBED, B_BLK), jnp.float32),
            pltpu.SemaphoreType.DMA((2,)),
            pltpu.SemaphoreType.DMA((2,)),
        ],
        compiler_params=pltpu.CompilerParams(use_tc_tiling_on_sc=True, needs_layout_passes=False),
    )(xt, wpk)
    return jnp.transpose(out, (2, 0, 1))


# speedup vs baseline: 1.2521x; 1.2521x over previous
import functools

import jax
import jax.numpy as jnp
from jax import lax
from jax.experimental import pallas as pl
from jax.experimental.pallas import tpu as pltpu
from jax.experimental.pallas import tpu_sc as plsc

BATCH = 4096
HIST = 200
D_EMBED = 64
N_VOCAB = 1000000
NUM_CORES = 2
NUM_SUBCORES = 16
NW = NUM_CORES * NUM_SUBCORES
L = 16

TILE_COLS = N_VOCAB // 128
TAIL_R0 = TILE_COLS * 128
N_PAIR = N_VOCAB // 2

B_BLK = BATCH // NW


def _fmt_body(wt_hbm, tail_hbm, wpk_hbm, src_v, dst_v, tail_v, sem):
    wid = lax.axis_index("s") * NUM_CORES + lax.axis_index("c")

    lane = lax.iota(jnp.int32, L)
    lane_row = lane >> 1
    lane_col = (lane & 1) * 64

    rows_m = [m * 8 + lane_row for m in range(8)]
    cols_k = [lane_col + k for k in range(8)]

    def do_col(j):
        pltpu.sync_copy(wt_hbm.at[:, pl.ds(j * 128, 128)], src_v)

        def row_dd(dd, c):
            d0 = dd * 8
            for k in range(8):
                col = cols_k[k] + d0
                for m in range(8):
                    v = src_v[d0 + k, pl.ds(m * L, L)]
                    plsc.store_scatter(dst_v, [rows_m[m], col], v)
            return c

        lax.fori_loop(0, 8, row_dd, 0)
        pltpu.sync_copy(dst_v, wpk_hbm.at[pl.ds(j * 64, 64)])

    def step(t, carry):
        j = wid + t * NW

        @pl.when(j < TILE_COLS)
        def _():
            do_col(j)

        return carry

    n_steps = (TILE_COLS + NW - 1) // NW
    lax.fori_loop(0, n_steps, step, 0)

    @pl.when(wid == 0)
    def _():
        pltpu.sync_copy(tail_hbm, tail_v)

        def row_q(q, c):
            rowv = lane_row * 0 + (q >> 1)
            colb = (q & 1) * 64

            def seg_m(m, c2):
                v = tail_v[q, pl.ds(m * L, L)]
                plsc.store_scatter(dst_v, [rowv, colb + m * L + lane], v)
                return c2

            return lax.fori_loop(0, 4, seg_m, c, unroll=True)

        lax.fori_loop(0, 64, row_q, 0)
        pltpu.sync_copy(dst_v.at[pl.ds(0, 32)], wpk_hbm.at[pl.ds(N_PAIR - 32, 32)])


def _gather_body(xt_hbm, wpk_hbm, out_hbm, idx_v, par_v, gbuf_v, obuf_v,
                 gsem, osem):
    wid = lax.axis_index("s") * NUM_CORES + lax.axis_index("c")
    b0 = wid * B_BLK

    pltpu.sync_copy(xt_hbm.at[:, pl.ds(b0, B_BLK)], idx_v)

    lane = lax.iota(jnp.int32, L)
    rows_m = [m * L + lane for m in range(8)]

    def conv_h(h, c):
        for m in range(8):
            r = idx_v[h, pl.ds(m * L, L)]
            par_v[h, pl.ds(m * L, L)] = (r & 1) * 64
            idx_v[h, pl.ds(m * L, L)] = r >> 1
        return c

    lax.fori_loop(0, HIST, conv_h, 0)

    def wait_g(slot):
        pltpu.make_async_copy(
            wpk_hbm.at[idx_v.at[0]], gbuf_v.at[slot], gsem.at[slot]
        ).wait()

    def start_g(h, slot):
        pltpu.async_copy(
            wpk_hbm.at[idx_v.at[h]], gbuf_v.at[slot], gsem.at[slot]
        )

    def wait_o(slot):
        pltpu.make_async_copy(
            obuf_v.at[slot], out_hbm.at[0, :, pl.ds(b0, B_BLK)], osem.at[slot]
        ).wait()

    def start_o(h, slot):
        pltpu.async_copy(
            obuf_v.at[slot], out_hbm.at[h, :, pl.ds(b0, B_BLK)], osem.at[slot]
        )

    start_g(0, 0)
    start_g(1, 1)

    def step_hh(hh, carry):
        for slot in (0, 1):
            h = hh * 2 + slot
            wait_g(slot)

            @pl.when(h >= 2)
            def _():
                wait_o(slot)

            offs = [par_v[h, pl.ds(m * L, L)] for m in range(8)]

            def row_dd(dd, c):
                d0 = dd * 8
                for k in range(8):
                    for m in range(8):
                        obuf_v[slot, d0 + k, pl.ds(m * L, L)] = plsc.load_gather(
                            gbuf_v.at[slot], [rows_m[m], offs[m] + (d0 + k)]
                        )
                return c

            lax.fori_loop(0, 8, row_dd, 0)
            start_o(h, slot)

            @pl.when(h + 2 < HIST)
            def _():
                start_g(h + 2, slot)

        return carry

    lax.fori_loop(0, HIST // 2, step_hh, 0)
    wait_o(0)
    wait_o(1)


def kernel(x, W_E):
    mesh = plsc.VectorSubcoreMesh(core_axis_name="c", subcore_axis_name="s")
    wt = W_E.T
    tail = W_E[TAIL_R0:]

    wpk = pl.kernel(
        _fmt_body,
        mesh=mesh,
        out_type=jax.ShapeDtypeStruct((N_PAIR, 128), jnp.float32),
        scratch_types=[
            pltpu.VMEM((64, 128), jnp.float32),
            pltpu.VMEM((64, 128), jnp.float32),
            pltpu.VMEM((64, 64), jnp.float32),
            pltpu.SemaphoreType.DMA,
        ],
        compiler_params=pltpu.CompilerParams(use_tc_tiling_on_sc=True, needs_layout_passes=False),
    )(wt, tail)

    xt = x.T.astype(jnp.int32)
    out = pl.kernel(
        _gather_body,
        mesh=mesh,
        out_type=jax.ShapeDtypeStruct((HIST, D_EMBED, BATCH), jnp.float32),
        scratch_types=[
            pltpu.VMEM((HIST, B_BLK), jnp.int32),
            pltpu.VMEM((HIST, B_BLK), jnp.int32),
            pltpu.VMEM((2, B_BLK, 128), jnp.float32),
            pltpu.VMEM((2, D_EMBED, B_BLK), jnp.float32),
            pltpu.SemaphoreType.DMA((2,)),
            pltpu.SemaphoreType.DMA((2,)),
        ],
        compiler_params=pltpu.CompilerParams(use_tc_tiling_on_sc=True, needs_layout_passes=False),
    )(xt, wpk)
    return jnp.transpose(out, (2, 0, 1))


# speedup vs baseline: 2.6608x; 2.1251x over previous
import functools

import jax
import jax.numpy as jnp
from jax import lax
from jax.experimental import pallas as pl
from jax.experimental.pallas import tpu as pltpu
from jax.experimental.pallas import tpu_sc as plsc

BATCH = 4096
HIST = 200
D_EMBED = 64
N_VOCAB = 1000000
NUM_CORES = 2
NUM_SUBCORES = 16
NW = NUM_CORES * NUM_SUBCORES
L = 16

TILE_COLS = N_VOCAB // 128
TAIL_R0 = TILE_COLS * 128
N_PAIR = N_VOCAB // 2
WPK_FLAT = N_VOCAB * D_EMBED

B_BLK = BATCH // NW


def _fmt_body(wt_hbm, tail_hbm, wpk_hbm, src_v, dst_v, tail_v, sem):
    wid = lax.axis_index("s") * NUM_CORES + lax.axis_index("c")

    lane = lax.iota(jnp.int32, L)
    cols_q = [lane + 16 * q for q in range(8)]
    base_q = [lane * 64 + (16 * q) * 64 for q in range(8)]

    def do_col(j):
        pltpu.sync_copy(wt_hbm.at[:, pl.ds(j * 128, 128)], src_v)

        def kloop(k, c):
            dvec = (k + lane) & 63
            for q in range(8):
                v = plsc.load_gather(src_v, [dvec, cols_q[q]])
                plsc.store_scatter(dst_v, [base_q[q] + dvec], v)
            return c

        lax.fori_loop(0, 64, kloop, 0)
        pltpu.sync_copy(dst_v, wpk_hbm.at[pl.ds(j * 8192, 8192)])

    def step(t, carry):
        j = wid + t * NW

        @pl.when(j < TILE_COLS)
        def _():
            do_col(j)

        return carry

    n_steps = (TILE_COLS + NW - 1) // NW
    lax.fori_loop(0, n_steps, step, 0)

    @pl.when(wid == 0)
    def _():
        pltpu.sync_copy(tail_hbm, tail_v)
        pltpu.sync_copy(tail_v, wpk_hbm.at[pl.ds(WPK_FLAT - 4096, 4096)])


def _gather_body(xt_hbm, wpk_hbm, out_hbm, idx_v, par_v, gbuf_v, obuf_v,
                 gsem, osem):
    wid = lax.axis_index("s") * NUM_CORES + lax.axis_index("c")
    b0 = wid * B_BLK

    pltpu.sync_copy(xt_hbm.at[:, pl.ds(b0, B_BLK)], idx_v)

    lane = lax.iota(jnp.int32, L)
    rows_m = [m * L + lane for m in range(8)]

    def conv_h(h, c):
        for m in range(8):
            r = idx_v[h, pl.ds(m * L, L)]
            par_v[h, pl.ds(m * L, L)] = (r & 1) * 64
            idx_v[h, pl.ds(m * L, L)] = r >> 1
        return c

    lax.fori_loop(0, HIST, conv_h, 0)

    def wait_g(slot):
        pltpu.make_async_copy(
            wpk_hbm.at[idx_v.at[0]], gbuf_v.at[slot], gsem.at[slot]
        ).wait()

    def start_g(h, slot):
        pltpu.async_copy(
            wpk_hbm.at[idx_v.at[h]], gbuf_v.at[slot], gsem.at[slot]
        )

    def wait_o(slot):
        pltpu.make_async_copy(
            obuf_v.at[slot], out_hbm.at[0, :, pl.ds(b0, B_BLK)], osem.at[slot]
        ).wait()

    def start_o(h, slot):
        pltpu.async_copy(
            obuf_v.at[slot], out_hbm.at[h, :, pl.ds(b0, B_BLK)], osem.at[slot]
        )

    start_g(0, 0)
    start_g(1, 1)

    def step_hh(hh, carry):
        for slot in (0, 1):
            h = hh * 2 + slot
            wait_g(slot)

            @pl.when(h >= 2)
            def _():
                wait_o(slot)

            offs = [par_v[h, pl.ds(m * L, L)] for m in range(8)]

            def kloop(k, c):
                dvec = (k + lane) & 63
                for m in range(8):
                    v = plsc.load_gather(
                        gbuf_v.at[slot], [rows_m[m], offs[m] + dvec]
                    )
                    plsc.store_scatter(
                        obuf_v.at[slot], [dvec, rows_m[m]], v
                    )
                return c

            lax.fori_loop(0, 64, kloop, 0)
            start_o(h, slot)

            @pl.when(h + 2 < HIST)
            def _():
                start_g(h + 2, slot)

        return carry

    lax.fori_loop(0, HIST // 2, step_hh, 0)
    wait_o(0)
    wait_o(1)


def kernel(x, W_E):
    mesh = plsc.VectorSubcoreMesh(core_axis_name="c", subcore_axis_name="s")
    wt = W_E.T
    tail = W_E[TAIL_R0:].reshape(-1)

    wpk = pl.kernel(
        _fmt_body,
        mesh=mesh,
        out_type=jax.ShapeDtypeStruct((WPK_FLAT,), jnp.float32),
        scratch_types=[
            pltpu.VMEM((64, 128), jnp.float32),
            pltpu.VMEM((8192,), jnp.float32),
            pltpu.VMEM((4096,), jnp.float32),
            pltpu.SemaphoreType.DMA,
        ],
        compiler_params=pltpu.CompilerParams(
            use_tc_tiling_on_sc=True, needs_layout_passes=False
        ),
    )(wt, tail)

    xt = x.T.astype(jnp.int32)
    out = pl.kernel(
        _gather_body,
        mesh=mesh,
        out_type=jax.ShapeDtypeStruct((HIST, D_EMBED, BATCH), jnp.float32),
        scratch_types=[
            pltpu.VMEM((HIST, B_BLK), jnp.int32),
            pltpu.VMEM((HIST, B_BLK), jnp.int32),
            pltpu.VMEM((2, B_BLK, 128), jnp.float32),
            pltpu.VMEM((2, D_EMBED, B_BLK), jnp.float32),
            pltpu.SemaphoreType.DMA((2,)),
            pltpu.SemaphoreType.DMA((2,)),
        ],
        compiler_params=pltpu.CompilerParams(
            use_tc_tiling_on_sc=True, needs_layout_passes=False
        ),
    )(xt, wpk.reshape(N_PAIR, 128))
    return jnp.transpose(out, (2, 0, 1))


# speedup vs baseline: 3.6383x; 1.3673x over previous
import functools

import jax
import jax.numpy as jnp
from jax import lax
from jax.experimental import pallas as pl
from jax.experimental.pallas import tpu as pltpu
from jax.experimental.pallas import tpu_sc as plsc

BATCH = 4096
HIST = 200
D_EMBED = 64
N_VOCAB = 1000000
NUM_CORES = 2
NUM_SUBCORES = 16
NW = NUM_CORES * NUM_SUBCORES
L = 16

TILE_COLS = N_VOCAB // 128
TAIL_R0 = TILE_COLS * 128
N_PAIR = N_VOCAB // 2
WPK_FLAT = N_VOCAB * D_EMBED

B_BLK = BATCH // NW


def _fmt_body(wt_hbm, tail_hbm, wpk_hbm, src0_v, src1_v, dst0_v, dst1_v, tail_v,
              isem, osem):
    wid = lax.axis_index("s") * NUM_CORES + lax.axis_index("c")

    lane = lax.iota(jnp.int32, L)
    cols_q = [lane + 16 * q for q in range(8)]
    base_q = [lane * 64 + (16 * q) * 64 for q in range(8)]

    srcs = (src0_v, src1_v)
    dsts = (dst0_v, dst1_v)

    def start_in(j, b):
        pltpu.async_copy(
            wt_hbm.at[:, pl.ds(j * 128, 128)], srcs[b], isem.at[b]
        )

    def wait_in(b):
        pltpu.make_async_copy(
            wt_hbm.at[:, pl.ds(0, 128)], srcs[b], isem.at[b]
        ).wait()

    def start_out(j, b):
        pltpu.async_copy(
            dsts[b], wpk_hbm.at[pl.ds(j * 8192, 8192)], osem.at[b]
        )

    def wait_out(b):
        pltpu.make_async_copy(
            dsts[b], wpk_hbm.at[pl.ds(0, 8192)], osem.at[b]
        ).wait()

    n_steps = (TILE_COLS + NW - 1) // NW
    start_in(wid, 0)

    def step_tt(tt, carry):
        for s in (0, 1):
            t = tt * 2 + s
            j = wid + t * NW

            @pl.when(j < TILE_COLS)
            def _():
                jn = wid + (t + 1) * NW

                @pl.when(jn < TILE_COLS)
                def _():
                    start_in(jn, 1 - s)

                wait_in(s)

                @pl.when(t >= 2)
                def _():
                    wait_out(s)

                def kloop(k, c):
                    dvec = (k + lane) & 63
                    for q in range(8):
                        v = plsc.load_gather(srcs[s], [dvec, cols_q[q]])
                        plsc.store_scatter(dsts[s], [base_q[q] + dvec], v)
                    return c

                lax.fori_loop(0, 64, kloop, 0)
                start_out(j, s)

        return carry

    lax.fori_loop(0, (n_steps + 1) // 2, step_tt, 0)
    wait_out(0)
    wait_out(1)

    @pl.when(wid == 0)
    def _():
        pltpu.sync_copy(tail_hbm, tail_v)
        pltpu.sync_copy(tail_v, wpk_hbm.at[pl.ds(WPK_FLAT - 4096, 4096)])


def _gather_body(xt_hbm, wpk_hbm, out_hbm, idx_v, par_v, gbuf_v, obuf_v,
                 gsem, osem):
    wid = lax.axis_index("s") * NUM_CORES + lax.axis_index("c")
    b0 = wid * B_BLK

    pltpu.sync_copy(xt_hbm.at[:, pl.ds(b0, B_BLK)], idx_v)

    lane = lax.iota(jnp.int32, L)
    rows_m = [m * L + lane for m in range(8)]

    def conv_h(h, c):
        for m in range(8):
            r = idx_v[h, pl.ds(m * L, L)]
            par_v[h, pl.ds(m * L, L)] = (r & 1) * 64
            idx_v[h, pl.ds(m * L, L)] = r >> 1
        return c

    lax.fori_loop(0, HIST, conv_h, 0)

    def wait_g(slot):
        pltpu.make_async_copy(
            wpk_hbm.at[idx_v.at[0]], gbuf_v.at[slot], gsem.at[slot]
        ).wait()

    def start_g(h, slot):
        pltpu.async_copy(
            wpk_hbm.at[idx_v.at[h]], gbuf_v.at[slot], gsem.at[slot]
        )

    def wait_o(slot):
        pltpu.make_async_copy(
            obuf_v.at[slot], out_hbm.at[0, :, pl.ds(b0, B_BLK)], osem.at[slot]
        ).wait()

    def start_o(h, slot):
        pltpu.async_copy(
            obuf_v.at[slot], out_hbm.at[h, :, pl.ds(b0, B_BLK)], osem.at[slot]
        )

    start_g(0, 0)
    start_g(1, 1)

    def step_hh(hh, carry):
        for slot in (0, 1):
            h = hh * 2 + slot
            wait_g(slot)

            @pl.when(h >= 2)
            def _():
                wait_o(slot)

            offs = [par_v[h, pl.ds(m * L, L)] for m in range(8)]

            def kloop(k, c):
                dvec = (k + lane) & 63
                for m in range(8):
                    v = plsc.load_gather(
                        gbuf_v.at[slot], [rows_m[m], offs[m] + dvec]
                    )
                    plsc.store_scatter(
                        obuf_v.at[slot], [dvec, rows_m[m]], v
                    )
                return c

            lax.fori_loop(0, 64, kloop, 0)
            start_o(h, slot)

            @pl.when(h + 2 < HIST)
            def _():
                start_g(h + 2, slot)

        return carry

    lax.fori_loop(0, HIST // 2, step_hh, 0)
    wait_o(0)
    wait_o(1)


def kernel(x, W_E):
    mesh = plsc.VectorSubcoreMesh(core_axis_name="c", subcore_axis_name="s")
    wt = W_E.T
    tail = W_E[TAIL_R0:].reshape(-1)

    wpk = pl.kernel(
        _fmt_body,
        mesh=mesh,
        out_type=jax.ShapeDtypeStruct((WPK_FLAT,), jnp.float32),
        scratch_types=[
            pltpu.VMEM((64, 128), jnp.float32),
            pltpu.VMEM((64, 128), jnp.float32),
            pltpu.VMEM((8192,), jnp.float32),
            pltpu.VMEM((8192,), jnp.float32),
            pltpu.VMEM((4096,), jnp.float32),
            pltpu.SemaphoreType.DMA((2,)),
            pltpu.SemaphoreType.DMA((2,)),
        ],
        compiler_params=pltpu.CompilerParams(
            use_tc_tiling_on_sc=True, needs_layout_passes=False
        ),
    )(wt, tail)

    xt = x.T.astype(jnp.int32)
    out = pl.kernel(
        _gather_body,
        mesh=mesh,
        out_type=jax.ShapeDtypeStruct((HIST, D_EMBED, BATCH), jnp.float32),
        scratch_types=[
            pltpu.VMEM((HIST, B_BLK), jnp.int32),
            pltpu.VMEM((HIST, B_BLK), jnp.int32),
            pltpu.VMEM((2, B_BLK, 128), jnp.float32),
            pltpu.VMEM((2, D_EMBED, B_BLK), jnp.float32),
            pltpu.SemaphoreType.DMA((2,)),
            pltpu.SemaphoreType.DMA((2,)),
        ],
        compiler_params=pltpu.CompilerParams(
            use_tc_tiling_on_sc=True, needs_layout_passes=False
        ),
    )(xt, wpk.reshape(N_PAIR, 128))
    return jnp.transpose(out, (2, 0, 1))


# speedup vs baseline: 3.6507x; 1.0034x over previous
import functools

import jax
import jax.numpy as jnp
from jax import lax
from jax.experimental import pallas as pl
from jax.experimental.pallas import tpu as pltpu
from jax.experimental.pallas import tpu_sc as plsc

BATCH = 4096
HIST = 200
D_EMBED = 64
N_VOCAB = 1000000
NUM_CORES = 2
NUM_SUBCORES = 16
NW = NUM_CORES * NUM_SUBCORES
L = 16

TILE_COLS = N_VOCAB // 128
TAIL_R0 = TILE_COLS * 128
N_PAIR = N_VOCAB // 2
WPK_FLAT = N_VOCAB * D_EMBED

B_BLK = BATCH // NW


def _fmt_body(wt_hbm, tail_hbm, wpk_hbm, src0_v, src1_v, dst0_v, dst1_v, tail_v,
              isem, osem):
    wid = lax.axis_index("s") * NUM_CORES + lax.axis_index("c")

    lane = lax.iota(jnp.int32, L)
    cols_q = [lane + 16 * q + 128 * half
              for half in range(2) for q in range(8)]
    base_q = [lane * 64 + (16 * q + 128 * half) * 64
              for half in range(2) for q in range(8)]

    srcs = (src0_v, src1_v)
    dsts = (dst0_v, dst1_v)

    def start_in(j2, b):
        pltpu.async_copy(
            wt_hbm.at[:, pl.ds(j2 * 256, 256)], srcs[b], isem.at[b]
        )

    def wait_in(b):
        pltpu.make_async_copy(
            wt_hbm.at[:, pl.ds(0, 256)], srcs[b], isem.at[b]
        ).wait()

    def start_out(j2, b):
        pltpu.async_copy(
            dsts[b], wpk_hbm.at[pl.ds(j2 * 16384, 16384)], osem.at[b]
        )

    def wait_out(b):
        pltpu.make_async_copy(
            dsts[b], wpk_hbm.at[pl.ds(0, 16384)], osem.at[b]
        ).wait()

    N_PAIRCOLS = TILE_COLS // 2
    n_steps = (N_PAIRCOLS + NW - 1) // NW
    start_in(wid, 0)

    def step_tt(tt, carry):
        for s in (0, 1):
            t = tt * 2 + s
            j2 = wid + t * NW

            @pl.when(j2 < N_PAIRCOLS)
            def _():
                j2n = wid + (t + 1) * NW

                @pl.when(j2n < N_PAIRCOLS)
                def _():
                    start_in(j2n, 1 - s)

                wait_in(s)

                @pl.when(t >= 2)
                def _():
                    wait_out(s)

                def kloop(k, c):
                    dvec = (k + lane) & 63
                    for qq in range(16):
                        v = plsc.load_gather(srcs[s], [dvec, cols_q[qq]])
                        plsc.store_scatter(dsts[s], [base_q[qq] + dvec], v)
                    return c

                lax.fori_loop(0, 64, kloop, 0)
                start_out(j2, s)

        return carry

    lax.fori_loop(0, (n_steps + 1) // 2, step_tt, 0)
    wait_out(0)
    wait_out(1)

    @pl.when(wid == 0)
    def _():
        pltpu.sync_copy(tail_hbm, tail_v)
        pltpu.sync_copy(tail_v, wpk_hbm.at[pl.ds(WPK_FLAT - 4096, 4096)])


def _gather_body(xt_hbm, wpk_hbm, out_hbm, idx_v, par_v, gbuf_v, obuf_v,
                 gsem, osem):
    wid = lax.axis_index("s") * NUM_CORES + lax.axis_index("c")
    b0 = wid * B_BLK

    pltpu.sync_copy(xt_hbm.at[:, pl.ds(b0, B_BLK)], idx_v)

    lane = lax.iota(jnp.int32, L)
    rows_m = [m * L + lane for m in range(8)]

    def conv_h(h, c):
        for m in range(8):
            r = idx_v[h, pl.ds(m * L, L)]
            par_v[h, pl.ds(m * L, L)] = (r & 1) * 64
            idx_v[h, pl.ds(m * L, L)] = r >> 1
        return c

    lax.fori_loop(0, HIST, conv_h, 0)

    def wait_g(slot):
        pltpu.make_async_copy(
            wpk_hbm.at[idx_v.at[0]], gbuf_v.at[slot], gsem.at[slot]
        ).wait()

    def start_g(h, slot):
        pltpu.async_copy(
            wpk_hbm.at[idx_v.at[h]], gbuf_v.at[slot], gsem.at[slot]
        )

    def wait_o(slot):
        pltpu.make_async_copy(
            obuf_v.at[slot], out_hbm.at[0, :, pl.ds(b0, B_BLK)], osem.at[slot]
        ).wait()

    def start_o(h, slot):
        pltpu.async_copy(
            obuf_v.at[slot], out_hbm.at[h, :, pl.ds(b0, B_BLK)], osem.at[slot]
        )

    start_g(0, 0)
    start_g(1, 1)

    def step_hh(hh, carry):
        for slot in (0, 1):
            h = hh * 2 + slot
            wait_g(slot)

            @pl.when(h >= 2)
            def _():
                wait_o(slot)

            offs = [par_v[h, pl.ds(m * L, L)] for m in range(8)]

            def kloop(k, c):
                dvec = (k + lane) & 63
                for m in range(8):
                    v = plsc.load_gather(
                        gbuf_v.at[slot], [rows_m[m], offs[m] + dvec]
                    )
                    plsc.store_scatter(
                        obuf_v.at[slot], [dvec, rows_m[m]], v
                    )
                return c

            lax.fori_loop(0, 64, kloop, 0)
            start_o(h, slot)

            @pl.when(h + 2 < HIST)
            def _():
                start_g(h + 2, slot)

        return carry

    lax.fori_loop(0, HIST // 2, step_hh, 0)
    wait_o(0)
    wait_o(1)


def kernel(x, W_E):
    mesh = plsc.VectorSubcoreMesh(core_axis_name="c", subcore_axis_name="s")
    wt = W_E.T
    tail = W_E[TAIL_R0:].reshape(-1)

    wpk = pl.kernel(
        _fmt_body,
        mesh=mesh,
        out_type=jax.ShapeDtypeStruct((WPK_FLAT,), jnp.float32),
        scratch_types=[
            pltpu.VMEM((64, 256), jnp.float32),
            pltpu.VMEM((64, 256), jnp.float32),
            pltpu.VMEM((16384,), jnp.float32),
            pltpu.VMEM((16384,), jnp.float32),
            pltpu.VMEM((4096,), jnp.float32),
            pltpu.SemaphoreType.DMA((2,)),
            pltpu.SemaphoreType.DMA((2,)),
        ],
        compiler_params=pltpu.CompilerParams(
            use_tc_tiling_on_sc=True, needs_layout_passes=False
        ),
    )(wt, tail)

    xt = x.T.astype(jnp.int32)
    out = pl.kernel(
        _gather_body,
        mesh=mesh,
        out_type=jax.ShapeDtypeStruct((HIST, D_EMBED, BATCH), jnp.float32),
        scratch_types=[
            pltpu.VMEM((HIST, B_BLK), jnp.int32),
            pltpu.VMEM((HIST, B_BLK), jnp.int32),
            pltpu.VMEM((2, B_BLK, 128), jnp.float32),
            pltpu.VMEM((2, D_EMBED, B_BLK), jnp.float32),
            pltpu.SemaphoreType.DMA((2,)),
            pltpu.SemaphoreType.DMA((2,)),
        ],
        compiler_params=pltpu.CompilerParams(
            use_tc_tiling_on_sc=True, needs_layout_passes=False
        ),
    )(xt, wpk.reshape(N_PAIR, 128))
    return jnp.transpose(out, (2, 0, 1))


# speedup vs baseline: 3.6608x; 1.0028x over previous
import functools

import jax
import jax.numpy as jnp
from jax import lax
from jax.experimental import pallas as pl
from jax.experimental.pallas import tpu as pltpu
from jax.experimental.pallas import tpu_sc as plsc

BATCH = 4096
HIST = 200
D_EMBED = 64
N_VOCAB = 1000000
NUM_CORES = 2
NUM_SUBCORES = 16
NW = NUM_CORES * NUM_SUBCORES
L = 16

TILE_COLS = N_VOCAB // 128
TAIL_R0 = TILE_COLS * 128
N_PAIR = N_VOCAB // 2
WPK_FLAT = N_VOCAB * D_EMBED

B_BLK = BATCH // NW


def _fmt_body(wt_hbm, tail_hbm, wpk_hbm, src0_v, src1_v, src2_v, dst0_v, dst1_v,
              dst2_v, tail_v, isem, osem):
    wid = lax.axis_index("s") * NUM_CORES + lax.axis_index("c")

    lane = lax.iota(jnp.int32, L)
    cols_q = [lane + 16 * q + 128 * half
              for half in range(2) for q in range(8)]
    base_q = [lane * 64 + (16 * q + 128 * half) * 64
              for half in range(2) for q in range(8)]

    srcs = (src0_v, src1_v, src2_v)
    dsts = (dst0_v, dst1_v, dst2_v)

    def start_in(j2, b):
        pltpu.async_copy(
            wt_hbm.at[:, pl.ds(j2 * 256, 256)], srcs[b], isem.at[b]
        )

    def wait_in(b):
        pltpu.make_async_copy(
            wt_hbm.at[:, pl.ds(0, 256)], srcs[b], isem.at[b]
        ).wait()

    def start_out(j2, b):
        pltpu.async_copy(
            dsts[b], wpk_hbm.at[pl.ds(j2 * 16384, 16384)], osem.at[b]
        )

    def wait_out(b):
        pltpu.make_async_copy(
            dsts[b], wpk_hbm.at[pl.ds(0, 16384)], osem.at[b]
        ).wait()

    N_PAIRCOLS = TILE_COLS // 2
    n_steps = (N_PAIRCOLS + NW - 1) // NW
    start_in(wid, 0)
    start_in(wid + NW, 1)

    def step_tt(tt, carry):
        for s in (0, 1, 2):
            t = tt * 3 + s
            j2 = wid + t * NW

            @pl.when(j2 < N_PAIRCOLS)
            def _():
                j2n = wid + (t + 2) * NW

                @pl.when(j2n < N_PAIRCOLS)
                def _():
                    start_in(j2n, (s + 2) % 3)

                wait_in(s)

                @pl.when(t >= 3)
                def _():
                    wait_out(s)

                def kloop(k, c):
                    dvec = (k + lane) & 63
                    for qq in range(16):
                        v = plsc.load_gather(srcs[s], [dvec, cols_q[qq]])
                        plsc.store_scatter(dsts[s], [base_q[qq] + dvec], v)
                    return c

                lax.fori_loop(0, 64, kloop, 0)
                start_out(j2, s)

        return carry

    lax.fori_loop(0, (n_steps + 2) // 3, step_tt, 0)
    wait_out(0)
    wait_out(1)
    wait_out(2)

    @pl.when(wid == 0)
    def _():
        pltpu.sync_copy(tail_hbm, tail_v)
        pltpu.sync_copy(tail_v, wpk_hbm.at[pl.ds(WPK_FLAT - 4096, 4096)])


def _gather_body(xt_hbm, wpk_hbm, out_hbm, idx_v, par_v, gbuf_v, obuf_v,
                 gsem, osem):
    wid = lax.axis_index("s") * NUM_CORES + lax.axis_index("c")
    b0 = wid * B_BLK

    pltpu.sync_copy(xt_hbm.at[:, pl.ds(b0, B_BLK)], idx_v)

    lane = lax.iota(jnp.int32, L)
    rows_m = [m * L + lane for m in range(8)]

    def conv_h(h, c):
        for m in range(8):
            r = idx_v[h, pl.ds(m * L, L)]
            par_v[h, pl.ds(m * L, L)] = (r & 1) * 64
            idx_v[h, pl.ds(m * L, L)] = r >> 1
        return c

    lax.fori_loop(0, HIST, conv_h, 0)

    def wait_g(slot):
        pltpu.make_async_copy(
            wpk_hbm.at[idx_v.at[0]], gbuf_v.at[slot], gsem.at[slot]
        ).wait()

    def start_g(h, slot):
        pltpu.async_copy(
            wpk_hbm.at[idx_v.at[h]], gbuf_v.at[slot], gsem.at[slot]
        )

    def wait_o(slot):
        pltpu.make_async_copy(
            obuf_v.at[slot], out_hbm.at[0, :, pl.ds(b0, B_BLK)], osem.at[slot]
        ).wait()

    def start_o(h, slot):
        pltpu.async_copy(
            obuf_v.at[slot], out_hbm.at[h, :, pl.ds(b0, B_BLK)], osem.at[slot]
        )

    start_g(0, 0)
    start_g(1, 1)

    def step_hh(hh, carry):
        for slot in (0, 1):
            h = hh * 2 + slot
            wait_g(slot)

            @pl.when(h >= 2)
            def _():
                wait_o(slot)

            offs = [par_v[h, pl.ds(m * L, L)] for m in range(8)]

            def kloop(k, c):
                dvec = (k + lane) & 63
                for m in range(8):
                    v = plsc.load_gather(
                        gbuf_v.at[slot], [rows_m[m], offs[m] + dvec]
                    )
                    plsc.store_scatter(
                        obuf_v.at[slot], [dvec, rows_m[m]], v
                    )
                return c

            lax.fori_loop(0, 64, kloop, 0)
            start_o(h, slot)

            @pl.when(h + 2 < HIST)
            def _():
                start_g(h + 2, slot)

        return carry

    lax.fori_loop(0, HIST // 2, step_hh, 0)
    wait_o(0)
    wait_o(1)


def kernel(x, W_E):
    mesh = plsc.VectorSubcoreMesh(core_axis_name="c", subcore_axis_name="s")
    wt = W_E.T
    tail = W_E[TAIL_R0:].reshape(-1)

    wpk = pl.kernel(
        _fmt_body,
        mesh=mesh,
        out_type=jax.ShapeDtypeStruct((WPK_FLAT,), jnp.float32),
        scratch_types=[
            pltpu.VMEM((64, 256), jnp.float32),
            pltpu.VMEM((64, 256), jnp.float32),
            pltpu.VMEM((64, 256), jnp.float32),
            pltpu.VMEM((16384,), jnp.float32),
            pltpu.VMEM((16384,), jnp.float32),
            pltpu.VMEM((16384,), jnp.float32),
            pltpu.VMEM((4096,), jnp.float32),
            pltpu.SemaphoreType.DMA((3,)),
            pltpu.SemaphoreType.DMA((3,)),
        ],
        compiler_params=pltpu.CompilerParams(
            use_tc_tiling_on_sc=True, needs_layout_passes=False
        ),
    )(wt, tail)

    xt = x.T.astype(jnp.int32)
    out = pl.kernel(
        _gather_body,
        mesh=mesh,
        out_type=jax.ShapeDtypeStruct((HIST, D_EMBED, BATCH), jnp.float32),
        scratch_types=[
            pltpu.VMEM((HIST, B_BLK), jnp.int32),
            pltpu.VMEM((HIST, B_BLK), jnp.int32),
            pltpu.VMEM((2, B_BLK, 128), jnp.float32),
            pltpu.VMEM((2, D_EMBED, B_BLK), jnp.float32),
            pltpu.SemaphoreType.DMA((2,)),
            pltpu.SemaphoreType.DMA((2,)),
        ],
        compiler_params=pltpu.CompilerParams(
            use_tc_tiling_on_sc=True, needs_layout_passes=False
        ),
    )(xt, wpk.reshape(N_PAIR, 128))
    return jnp.transpose(out, (2, 0, 1))


# speedup vs baseline: 3.6675x; 1.0018x over previous
import jax
import jax.numpy as jnp
from jax import lax
from jax.experimental import pallas as pl
from jax.experimental.pallas import tpu as pltpu
from jax.experimental.pallas import tpu_sc as plsc

BATCH = 4096
HIST = 200
D_EMBED = 64
N_VOCAB = 1000000
NUM_CORES = 2
NUM_SUBCORES = 16
NW = NUM_CORES * NUM_SUBCORES
L = 16

TILE_COLS = N_VOCAB // 128
TAIL_R0 = TILE_COLS * 128
N_PAIR = N_VOCAB // 2
WPK_FLAT = N_VOCAB * D_EMBED

B_BLK = BATCH // NW


def _fmt_body(wt_hbm, tail_hbm, wpk_hbm, src0_v, src1_v, src2_v, dst0_v, dst1_v,
              dst2_v, tail_v, isem, osem):
    wid = lax.axis_index("s") * NUM_CORES + lax.axis_index("c")

    lane = lax.iota(jnp.int32, L)
    cols_q = [lane + 16 * q + 128 * half
              for half in range(2) for q in range(8)]
    base_q = [lane * 64 + (16 * q + 128 * half) * 64
              for half in range(2) for q in range(8)]

    srcs = (src0_v, src1_v, src2_v)
    dsts = (dst0_v, dst1_v, dst2_v)

    def start_in(j2, b):
        pltpu.async_copy(
            wt_hbm.at[:, pl.ds(j2 * 256, 256)], srcs[b], isem.at[b]
        )

    def wait_in(b):
        pltpu.make_async_copy(
            wt_hbm.at[:, pl.ds(0, 256)], srcs[b], isem.at[b]
        ).wait()

    def start_out(j2, b):
        pltpu.async_copy(
            dsts[b], wpk_hbm.at[pl.ds(j2 * 16384, 16384)], osem.at[b]
        )

    def wait_out(b):
        pltpu.make_async_copy(
            dsts[b], wpk_hbm.at[pl.ds(0, 16384)], osem.at[b]
        ).wait()

    N_PAIRCOLS = TILE_COLS // 2
    n_steps = (N_PAIRCOLS + NW - 1) // NW
    start_in(wid, 0)
    start_in(wid + NW, 1)

    def step_tt(tt, carry):
        for s in (0, 1, 2):
            t = tt * 3 + s
            j2 = wid + t * NW

            @pl.when(j2 < N_PAIRCOLS)
            def _():
                j2n = wid + (t + 2) * NW

                @pl.when(j2n < N_PAIRCOLS)
                def _():
                    start_in(j2n, (s + 2) % 3)

                wait_in(s)

                @pl.when(t >= 3)
                def _():
                    wait_out(s)

                def kloop(k, c):
                    dvec = (k + lane) & 63
                    for qq in range(16):
                        v = plsc.load_gather(srcs[s], [dvec, cols_q[qq]])
                        plsc.store_scatter(dsts[s], [base_q[qq] + dvec], v)
                    return c

                lax.fori_loop(0, 64, kloop, 0)
                start_out(j2, s)

        return carry

    lax.fori_loop(0, (n_steps + 2) // 3, step_tt, 0)
    wait_out(0)
    wait_out(1)
    wait_out(2)

    @pl.when(wid == 0)
    def _():
        pltpu.sync_copy(tail_hbm, tail_v)
        pltpu.sync_copy(tail_v, wpk_hbm.at[pl.ds(WPK_FLAT - 4096, 4096)])


def _gather_body(xt_hbm, wpk_hbm, out_hbm, idx_v, par_v, gbuf_v, obuf_v,
                 gsem, osem):
    wid = lax.axis_index("s") * NUM_CORES + lax.axis_index("c")
    b0 = wid * B_BLK

    pltpu.sync_copy(xt_hbm.at[:, pl.ds(b0, B_BLK)], idx_v)

    lane = lax.iota(jnp.int32, L)
    rows_m = [m * L + lane for m in range(8)]

    def conv_h(h, c):
        for m in range(8):
            r = idx_v[h, pl.ds(m * L, L)]
            par_v[h, pl.ds(m * L, L)] = (r & 1) * 64
            idx_v[h, pl.ds(m * L, L)] = r >> 1
        return c

    lax.fori_loop(0, HIST, conv_h, 0)

    def wait_g(slot):
        pltpu.make_async_copy(
            wpk_hbm.at[idx_v.at[0]], gbuf_v.at[slot], gsem.at[slot]
        ).wait()

    def start_g(h, slot):
        pltpu.async_copy(
            wpk_hbm.at[idx_v.at[h]], gbuf_v.at[slot], gsem.at[slot]
        )

    def wait_o(slot):
        pltpu.make_async_copy(
            obuf_v.at[slot], out_hbm.at[0, :, pl.ds(b0, B_BLK)], osem.at[slot]
        ).wait()

    def start_o(h, slot):
        pltpu.async_copy(
            obuf_v.at[slot], out_hbm.at[h, :, pl.ds(b0, B_BLK)], osem.at[slot]
        )

    start_g(0, 0)
    start_g(1, 1)

    def step_hh(hh, carry):
        for slot in (0, 1):
            h = hh * 2 + slot
            wait_g(slot)

            @pl.when(h >= 2)
            def _():
                wait_o(slot)

            offs = [par_v[h, pl.ds(m * L, L)] for m in range(8)]

            def kloop(k, c):
                dvec = (k + lane) & 63
                for m in range(8):
                    v = plsc.load_gather(
                        gbuf_v.at[slot], [rows_m[m], offs[m] + dvec]
                    )
                    plsc.store_scatter(
                        obuf_v.at[slot], [dvec, rows_m[m]], v
                    )
                return c

            lax.fori_loop(0, 64, kloop, 0)
            start_o(h, slot)

            @pl.when(h + 2 < HIST)
            def _():
                start_g(h + 2, slot)

        return carry

    lax.fori_loop(0, HIST // 2, step_hh, 0)
    wait_o(0)
    wait_o(1)


def kernel(x, W_E):
    mesh = plsc.VectorSubcoreMesh(core_axis_name="c", subcore_axis_name="s")
    wt = W_E.T
    tail = W_E[TAIL_R0:].reshape(-1)

    wpk = pl.kernel(
        _fmt_body,
        mesh=mesh,
        out_type=jax.ShapeDtypeStruct((WPK_FLAT,), jnp.float32),
        scratch_types=[
            pltpu.VMEM((64, 256), jnp.float32),
            pltpu.VMEM((64, 256), jnp.float32),
            pltpu.VMEM((64, 256), jnp.float32),
            pltpu.VMEM((16384,), jnp.float32),
            pltpu.VMEM((16384,), jnp.float32),
            pltpu.VMEM((16384,), jnp.float32),
            pltpu.VMEM((4096,), jnp.float32),
            pltpu.SemaphoreType.DMA((3,)),
            pltpu.SemaphoreType.DMA((3,)),
        ],
        compiler_params=pltpu.CompilerParams(
            use_tc_tiling_on_sc=True, needs_layout_passes=False
        ),
    )(wt, tail)

    xt = x.T.astype(jnp.int32)
    out = pl.kernel(
        _gather_body,
        mesh=mesh,
        out_type=jax.ShapeDtypeStruct((HIST, D_EMBED, BATCH), jnp.float32),
        scratch_types=[
            pltpu.VMEM((HIST, B_BLK), jnp.int32),
            pltpu.VMEM((HIST, B_BLK), jnp.int32),
            pltpu.VMEM((2, B_BLK, 128), jnp.float32),
            pltpu.VMEM((2, D_EMBED, B_BLK), jnp.float32),
            pltpu.SemaphoreType.DMA((2,)),
            pltpu.SemaphoreType.DMA((2,)),
        ],
        compiler_params=pltpu.CompilerParams(
            use_tc_tiling_on_sc=True, needs_layout_passes=False
        ),
    )(xt, wpk.reshape(N_PAIR, 128))
    return jnp.transpose(out, (2, 0, 1))
